# baseline (speedup 1.0000x reference)
import functools

import jax
import jax.numpy as jnp
from jax import lax
from jax.experimental import pallas as pl
from jax.experimental.pallas import tpu as pltpu
from jax.experimental.pallas import tpu_sc as plsc

_E = 16
_ROWS = 8192
_COLS = 2048
_INV_N = 1.0 / float(_ROWS * _COLS)
_ALPHA = 1.0 / 1000.0
_Z = 0.001

_SC_ROWS = 1536
_TC_ROWS = _ROWS - _SC_ROWS
_NTILES = 32
_TROWS = _SC_ROWS // _NTILES
_CHR = 8
_NCHUNK = _TROWS // _CHR
_STRIPS = _COLS // _E

_BLK = 256
_GRID = _TC_ROWS // (2 * _BLK)


def _tc_sum_body(a_ref, b_ref, out_ref, acc_ref):
    i = pl.program_id(0)

    @pl.when(i == 0)
    def _init():
        acc_ref[0] = 0.0

    acc_ref[0] += jnp.sum(a_ref[...]) + jnp.sum(b_ref[...])

    @pl.when(i == _GRID - 1)
    def _fin():
        ids = lax.broadcasted_iota(jnp.int32, (1, _E), 1)
        out_ref[...] = jnp.where(ids == 0, acc_ref[0], 0.0)


def _tc_partial_sum(wm_state):
    wm3d = wm_state.reshape(_ROWS // _BLK, _BLK, _COLS)
    return pl.pallas_call(
        _tc_sum_body,
        grid=(_GRID,),
        in_specs=[
            pl.BlockSpec((_BLK, _COLS), lambda i: (2 * i, 0)),
            pl.BlockSpec((1, _BLK, _COLS), lambda i: (2 * i + 1, 0, 0)),
        ],
        out_specs=pl.BlockSpec((1, _E), lambda i: (0, 0)),
        out_shape=jax.ShapeDtypeStruct((1, _E), jnp.float32),
        scratch_shapes=[pltpu.SMEM((1,), jnp.float32)],
    )(wm_state, wm3d)


_SC_MESH = plsc.VectorSubcoreMesh(core_axis_name="c", subcore_axis_name="s")


@functools.partial(
    pl.kernel,
    out_type=jax.ShapeDtypeStruct((_NTILES, _E), jnp.float32),
    mesh=_SC_MESH,
    compiler_params=pltpu.CompilerParams(
        needs_layout_passes=False, use_tc_tiling_on_sc=True),
    scratch_types=[
        pltpu.VMEM((_CHR, _COLS), jnp.float32),
        pltpu.VMEM((_CHR, _COLS), jnp.float32),
        pltpu.VMEM((_E,), jnp.float32),
        pltpu.SemaphoreType.DMA,
        pltpu.SemaphoreType.DMA,
    ],
)
def _sc_partial_sums(wm_hbm, parts_hbm, b0, b1, v_out, s0, s1):
    cid = lax.axis_index("c")
    sid = lax.axis_index("s")
    wid = sid * 2 + cid
    row0 = _TC_ROWS + wid * _TROWS

    bufs = (b0, b1)
    sems = (s0, s1)

    accs = (jnp.zeros((_E,), jnp.float32),) * 4
    pending = pltpu.async_copy(wm_hbm.at[pl.ds(row0, _CHR)], b0, s0)
    for k in range(_NCHUNK):
        nxt = None
        if k + 1 < _NCHUNK:
            nxt = pltpu.async_copy(
                wm_hbm.at[pl.ds(row0 + (k + 1) * _CHR, _CHR)],
                bufs[(k + 1) % 2], sems[(k + 1) % 2])
        pending.wait()
        buf = bufs[k % 2]

        for r in range(_CHR):
            def _body(j, accs, buf=buf, r=r):
                a0, a1, a2, a3 = accs
                off = j * (4 * _E)
                return (a0 + buf[r, pl.ds(off, _E)],
                        a1 + buf[r, pl.ds(off + _E, _E)],
                        a2 + buf[r, pl.ds(off + 2 * _E, _E)],
                        a3 + buf[r, pl.ds(off + 3 * _E, _E)])

            accs = lax.fori_loop(0, _STRIPS // 4, _body, accs)
        pending = nxt

    v_out[...] = (accs[0] + accs[1]) + (accs[2] + accs[3])
    pltpu.sync_copy(v_out, parts_hbm.at[wid])


def _combine_body(tc_ref, parts_ref, proto_ref, ema_ref,
                  mask_ref, probs_ref, loss_ref, idx_ref, usage_ref):
    total = jnp.sum(parts_ref[...]) + jnp.sum(tc_ref[...])
    x = total * _INV_N
    ids = lax.broadcasted_iota(jnp.int32, (1, _E), 1)
    p = proto_ref[...]
    sim = -((p - x) ** 2)
    m = jnp.max(sim)
    e = jnp.exp(sim - m)
    probs = e / jnp.sum(e)
    m1 = jnp.max(probs)
    i1 = jnp.min(jnp.where(probs == m1, ids, _E))
    hit1 = ids == i1
    probs2 = jnp.where(hit1, -jnp.inf, probs)
    m2 = jnp.max(probs2)
    i2 = jnp.min(jnp.where(probs2 == m2, ids, _E))
    mask = (hit1 | (ids == i2)).astype(jnp.float32)
    usage = (1.0 - _ALPHA) * ema_ref[...] + _ALPHA * mask
    d = usage - (1.0 / _E)
    loss = jnp.sum(d * d) * (1.0 / _E) * _Z
    mask_ref[...] = mask
    probs_ref[...] = probs
    loss_ref[...] = jnp.full((1, _E), loss, jnp.float32)
    idx_ref[...] = jnp.where(ids == 0, i1, jnp.where(ids == 1, i2, 0))
    usage_ref[...] = usage


def _combine(tc16, sc_parts, proto2d, ema2d):
    full = pl.BlockSpec((1, _E), lambda: (0, 0))
    return pl.pallas_call(
        _combine_body,
        in_specs=[full, pl.BlockSpec((_NTILES, _E), lambda: (0, 0)),
                  full, full],
        out_specs=[full, full, full, full, full],
        out_shape=[
            jax.ShapeDtypeStruct((1, _E), jnp.float32),
            jax.ShapeDtypeStruct((1, _E), jnp.float32),
            jax.ShapeDtypeStruct((1, _E), jnp.float32),
            jax.ShapeDtypeStruct((1, _E), jnp.int32),
            jax.ShapeDtypeStruct((1, _E), jnp.float32),
        ],
    )(tc16, sc_parts, proto2d, ema2d)


@jax.jit
def kernel(wm_state, prototypes, usage_ema):
    sc_parts = _sc_partial_sums(wm_state)
    tc16 = _tc_partial_sum(wm_state)
    mask2d, probs2d, loss2d, idx2d, usage2d = _combine(
        tc16, sc_parts, prototypes.reshape(1, _E), usage_ema.reshape(1, _E))
    return (mask2d[0], probs2d[0], loss2d[0, 0], idx2d[0, :2], usage2d[0])


# speedup vs baseline: 1.4924x; 1.4924x over previous
import jax
import jax.numpy as jnp
from jax import lax
from jax.experimental import pallas as pl
from jax.experimental.pallas import tpu as pltpu

_E = 16
_ROWS = 8192
_COLS = 2048
_BLK = 256
_GRID = _ROWS // (2 * _BLK)
_INV_N = 1.0 / float(_ROWS * _COLS)
_ALPHA = 1.0 / 1000.0
_Z = 0.001


def _router_kernel(a_ref, b_ref, proto_ref, ema_ref,
                   mask_ref, probs_ref, loss_ref, idx_ref, usage_ref,
                   acc_ref):
    i = pl.program_id(0)

    @pl.when(i == 0)
    def _init():
        acc_ref[0] = 0.0

    acc_ref[0] += jnp.sum(a_ref[...]) + jnp.sum(b_ref[...])

    @pl.when(i == _GRID - 1)
    def _epilogue():
        x = acc_ref[0] * _INV_N
        ids = lax.broadcasted_iota(jnp.int32, (1, _E), 1)
        p = proto_ref[...]
        sim = -((p - x) ** 2)
        m = jnp.max(sim)
        e = jnp.exp(sim - m)
        probs = e / jnp.sum(e)
        m1 = jnp.max(probs)
        i1 = jnp.min(jnp.where(probs == m1, ids, _E))
        hit1 = ids == i1
        probs2 = jnp.where(hit1, -jnp.inf, probs)
        m2 = jnp.max(probs2)
        i2 = jnp.min(jnp.where(probs2 == m2, ids, _E))
        mask = (hit1 | (ids == i2)).astype(jnp.float32)
        usage = (1.0 - _ALPHA) * ema_ref[...] + _ALPHA * mask
        d = usage - (1.0 / _E)
        loss = jnp.sum(d * d) * (1.0 / _E) * _Z
        mask_ref[...] = mask
        probs_ref[...] = probs
        loss_ref[...] = jnp.full((1, _E), loss, jnp.float32)
        idx_ref[...] = jnp.where(ids == 0, i1, jnp.where(ids == 1, i2, 0))
        usage_ref[...] = usage


@jax.jit
def kernel(wm_state, prototypes, usage_ema):
    wm3d = wm_state.reshape(_ROWS // _BLK, _BLK, _COLS)
    full = pl.BlockSpec((1, _E), lambda i: (0, 0))
    outs = pl.pallas_call(
        _router_kernel,
        grid=(_GRID,),
        in_specs=[
            pl.BlockSpec((_BLK, _COLS), lambda i: (2 * i, 0)),
            pl.BlockSpec((1, _BLK, _COLS), lambda i: (2 * i + 1, 0, 0)),
            full,
            full,
        ],
        out_specs=[full, full, full, full, full],
        out_shape=[
            jax.ShapeDtypeStruct((1, _E), jnp.float32),
            jax.ShapeDtypeStruct((1, _E), jnp.float32),
            jax.ShapeDtypeStruct((1, _E), jnp.float32),
            jax.ShapeDtypeStruct((1, _E), jnp.int32),
            jax.ShapeDtypeStruct((1, _E), jnp.float32),
        ],
        scratch_shapes=[pltpu.SMEM((1,), jnp.float32)],
    )(wm_state, wm3d, prototypes.reshape(1, _E), usage_ema.reshape(1, _E))
    mask2d, probs2d, loss2d, idx2d, usage2d = outs
    return (mask2d[0], probs2d[0], loss2d[0, 0], idx2d[0, :2], usage2d[0])


# speedup vs baseline: 1.6605x; 1.1126x over previous
import jax
import jax.numpy as jnp
from jax import lax
from jax.experimental import pallas as pl
from jax.experimental.pallas import tpu as pltpu

_E = 16
_ROWS = 8192
_COLS = 2048
_BLK = 1024
_GRID = _ROWS // (2 * _BLK)
_INV_N = 1.0 / float(_ROWS * _COLS)
_ALPHA = 1.0 / 1000.0
_Z = 0.001


def _router_kernel(a_ref, b_ref, proto_ref, ema_ref,
                   mask_ref, probs_ref, loss_ref, idx_ref, usage_ref,
                   acc_ref):
    i = pl.program_id(0)

    @pl.when(i == 0)
    def _init():
        acc_ref[0] = 0.0

    acc_ref[0] += jnp.sum(a_ref[...]) + jnp.sum(b_ref[...])

    @pl.when(i == _GRID - 1)
    def _epilogue():
        x = acc_ref[0] * _INV_N
        ids = lax.broadcasted_iota(jnp.int32, (1, _E), 1)
        p = proto_ref[...]
        sim = -((p - x) ** 2)
        m = jnp.max(sim)
        e = jnp.exp(sim - m)
        probs = e / jnp.sum(e)
        m1 = jnp.max(probs)
        i1 = jnp.min(jnp.where(probs == m1, ids, _E))
        hit1 = ids == i1
        probs2 = jnp.where(hit1, -jnp.inf, probs)
        m2 = jnp.max(probs2)
        i2 = jnp.min(jnp.where(probs2 == m2, ids, _E))
        mask = (hit1 | (ids == i2)).astype(jnp.float32)
        usage = (1.0 - _ALPHA) * ema_ref[...] + _ALPHA * mask
        d = usage - (1.0 / _E)
        loss = jnp.sum(d * d) * (1.0 / _E) * _Z
        mask_ref[...] = mask
        probs_ref[...] = probs
        loss_ref[...] = jnp.full((1, _E), loss, jnp.float32)
        idx_ref[...] = jnp.where(ids == 0, i1, jnp.where(ids == 1, i2, 0))
        usage_ref[...] = usage


@jax.jit
def kernel(wm_state, prototypes, usage_ema):
    wm3d = wm_state.reshape(_ROWS // _BLK, _BLK, _COLS)
    full = pl.BlockSpec((1, _E), lambda i: (0, 0))
    outs = pl.pallas_call(
        _router_kernel,
        grid=(_GRID,),
        in_specs=[
            pl.BlockSpec((_BLK, _COLS), lambda i: (2 * i, 0)),
            pl.BlockSpec((1, _BLK, _COLS), lambda i: (2 * i + 1, 0, 0)),
            full,
            full,
        ],
        out_specs=[full, full, full, full, full],
        out_shape=[
            jax.ShapeDtypeStruct((1, _E), jnp.float32),
            jax.ShapeDtypeStruct((1, _E), jnp.float32),
            jax.ShapeDtypeStruct((1, _E), jnp.float32),
            jax.ShapeDtypeStruct((1, _E), jnp.int32),
            jax.ShapeDtypeStruct((1, _E), jnp.float32),
        ],
        scratch_shapes=[pltpu.SMEM((1,), jnp.float32)],
    )(wm_state, wm3d, prototypes.reshape(1, _E), usage_ema.reshape(1, _E))
    mask2d, probs2d, loss2d, idx2d, usage2d = outs
    return (mask2d[0], probs2d[0], loss2d[0, 0], idx2d[0, :2], usage2d[0])


# speedup vs baseline: 1.6814x; 1.0126x over previous
import jax
import jax.numpy as jnp
from jax import lax
from jax.experimental import pallas as pl
from jax.experimental.pallas import tpu as pltpu

_E = 16
_ROWS = 8192
_COLS = 2048
_BLK = 512
_GRID = _ROWS // (2 * _BLK)
_INV_N = 1.0 / float(_ROWS * _COLS)
_ALPHA = 1.0 / 1000.0
_Z = 0.001


def _router_kernel(a_ref, b_ref, proto_ref, ema_ref,
                   mask_ref, probs_ref, loss_ref, idx_ref, usage_ref,
                   acc_ref):
    i = pl.program_id(0)

    @pl.when(i == 0)
    def _init():
        acc_ref[0] = 0.0

    acc_ref[0] += jnp.sum(a_ref[...]) + jnp.sum(b_ref[...])

    @pl.when(i == _GRID - 1)
    def _epilogue():
        x = acc_ref[0] * _INV_N
        ids = lax.broadcasted_iota(jnp.int32, (1, _E), 1)
        p = proto_ref[...]
        sim = -((p - x) ** 2)
        m = jnp.max(sim)
        e = jnp.exp(sim - m)
        probs = e / jnp.sum(e)
        m1 = jnp.max(probs)
        i1 = jnp.min(jnp.where(probs == m1, ids, _E))
        hit1 = ids == i1
        probs2 = jnp.where(hit1, -jnp.inf, probs)
        m2 = jnp.max(probs2)
        i2 = jnp.min(jnp.where(probs2 == m2, ids, _E))
        mask = (hit1 | (ids == i2)).astype(jnp.float32)
        usage = (1.0 - _ALPHA) * ema_ref[...] + _ALPHA * mask
        d = usage - (1.0 / _E)
        loss = jnp.sum(d * d) * (1.0 / _E) * _Z
        mask_ref[...] = mask
        probs_ref[...] = probs
        loss_ref[...] = jnp.full((1, _E), loss, jnp.float32)
        idx_ref[...] = jnp.where(ids == 0, i1, jnp.where(ids == 1, i2, 0))
        usage_ref[...] = usage


@jax.jit
def kernel(wm_state, prototypes, usage_ema):
    wm3d = wm_state.reshape(_ROWS // _BLK, _BLK, _COLS)
    full = pl.BlockSpec((1, _E), lambda i: (0, 0))
    outs = pl.pallas_call(
        _router_kernel,
        grid=(_GRID,),
        in_specs=[
            pl.BlockSpec((_BLK, _COLS), lambda i: (2 * i, 0)),
            pl.BlockSpec((1, _BLK, _COLS), lambda i: (2 * i + 1, 0, 0)),
            full,
            full,
        ],
        out_specs=[full, full, full, full, full],
        out_shape=[
            jax.ShapeDtypeStruct((1, _E), jnp.float32),
            jax.ShapeDtypeStruct((1, _E), jnp.float32),
            jax.ShapeDtypeStruct((1, _E), jnp.float32),
            jax.ShapeDtypeStruct((1, _E), jnp.int32),
            jax.ShapeDtypeStruct((1, _E), jnp.float32),
        ],
        scratch_shapes=[pltpu.SMEM((1,), jnp.float32)],
    )(wm_state, wm3d, prototypes.reshape(1, _E), usage_ema.reshape(1, _E))
    mask2d, probs2d, loss2d, idx2d, usage2d = outs
    return (mask2d[0], probs2d[0], loss2d[0, 0], idx2d[0, :2], usage2d[0])


# speedup vs baseline: 1.7023x; 1.0124x over previous
import jax
import jax.numpy as jnp
from jax import lax
from jax.experimental import pallas as pl
from jax.experimental.pallas import tpu as pltpu

_E = 16
_ROWS = 8192
_COLS = 2048
_BLK = 512
_GRID = _ROWS // (2 * _BLK)
_INV_N = 1.0 / float(_ROWS * _COLS)
_ALPHA = 1.0 / 1000.0
_Z = 0.001


def _router_kernel(a_ref, b_ref, proto_ref, ema_ref,
                   mask_ref, probs_ref, loss_ref, idx_ref, usage_ref,
                   acc_ref):
    i = pl.program_id(0)

    @pl.when(i == 0)
    def _init():
        acc_ref[0] = 0.0

    acc_ref[0] += jnp.sum(a_ref[...]) + jnp.sum(b_ref[...])

    @pl.when(i == _GRID - 1)
    def _epilogue():
        x = acc_ref[0] * _INV_N
        ids = lax.broadcasted_iota(jnp.int32, (1, _E), 1)
        p = proto_ref[...]
        sim = -((p - x) ** 2)
        m = jnp.max(sim)
        e = jnp.exp(sim - m)
        probs = e / jnp.sum(e)
        m1 = jnp.max(probs)
        i1 = jnp.min(jnp.where(probs == m1, ids, _E))
        hit1 = ids == i1
        probs2 = jnp.where(hit1, -jnp.inf, probs)
        m2 = jnp.max(probs2)
        i2 = jnp.min(jnp.where(probs2 == m2, ids, _E))
        mask = (hit1 | (ids == i2)).astype(jnp.float32)
        usage = (1.0 - _ALPHA) * ema_ref[...] + _ALPHA * mask
        d = usage - (1.0 / _E)
        loss = jnp.sum(d * d) * (1.0 / _E) * _Z
        mask_ref[...] = mask
        probs_ref[...] = probs
        loss_ref[...] = jnp.full((1, _E), loss, jnp.float32)
        idx_ref[...] = jnp.where(ids == 0, i1, jnp.where(ids == 1, i2, 0))
        usage_ref[...] = usage


@jax.jit
def kernel(wm_state, prototypes, usage_ema):
    wm3d = wm_state.reshape(_ROWS // _BLK, _BLK, _COLS)
    full = pl.BlockSpec((1, _E), lambda i: (0, 0))
    outs = pl.pallas_call(
        _router_kernel,
        grid=(_GRID,),
        in_specs=[
            pl.BlockSpec((_BLK, _COLS), lambda i: (i, 0)),
            pl.BlockSpec((1, _BLK, _COLS), lambda i: (i + _GRID, 0, 0)),
            full,
            full,
        ],
        out_specs=[full, full, full, full, full],
        out_shape=[
            jax.ShapeDtypeStruct((1, _E), jnp.float32),
            jax.ShapeDtypeStruct((1, _E), jnp.float32),
            jax.ShapeDtypeStruct((1, _E), jnp.float32),
            jax.ShapeDtypeStruct((1, _E), jnp.int32),
            jax.ShapeDtypeStruct((1, _E), jnp.float32),
        ],
        scratch_shapes=[pltpu.SMEM((1,), jnp.float32)],
    )(wm_state, wm3d, prototypes.reshape(1, _E), usage_ema.reshape(1, _E))
    mask2d, probs2d, loss2d, idx2d, usage2d = outs
    return (mask2d[0], probs2d[0], loss2d[0, 0], idx2d[0, :2], usage2d[0])
